# baseline (speedup 1.0000x reference)
import functools

import jax
import jax.numpy as jnp
from jax import lax
from jax.experimental import pallas as pl
from jax.experimental.pallas import tpu as pltpu
from jax.experimental.pallas import tpu_sc as plsc

NC = 2
NS = 16
NW = NC * NS
CHUNK = 80
BLK = 16
NBUF = 2
LANES = 16


def _matmul(x, W):
    n, d_in = x.shape
    d_out = W.shape[1]
    bm = 1000
    grid = (n // bm,)

    def body(x_ref, w_ref, o_ref):
        o_ref[...] = jnp.dot(x_ref[...], w_ref[...],
                             preferred_element_type=jnp.float32)

    return pl.pallas_call(
        body,
        grid=grid,
        in_specs=[
            pl.BlockSpec((bm, d_in), lambda i: (i, 0)),
            pl.BlockSpec((d_in, d_out), lambda i: (0, 0)),
        ],
        out_specs=pl.BlockSpec((bm, d_out), lambda i: (i, 0)),
        out_shape=jax.ShapeDtypeStruct((n, d_out), jnp.float32),
    )(x, W)


def _combine(partials):
    _, n, d = partials.shape
    bm = 1000
    grid = (n // bm,)

    def body(p_ref, o_ref):
        o_ref[...] = p_ref[0] + p_ref[1]

    return pl.pallas_call(
        body,
        grid=grid,
        in_specs=[pl.BlockSpec((2, bm, d), lambda i: (0, i, 0))],
        out_specs=pl.BlockSpec((bm, d), lambda i: (i, 0)),
        out_shape=jax.ShapeDtypeStruct((n, d), jnp.float32),
    )(partials)


def _sc_spmm(support, src4, dst4, ew3):
    n, d = support.shape
    nblk = src4.shape[1]
    rows_per_sub = (n // (8 * NS)) * 8
    n_rem = n - NS * rows_per_sub
    d_regs = d // LANES

    mesh = plsc.VectorSubcoreMesh(core_axis_name="c", subcore_axis_name="s")

    @functools.partial(
        pl.kernel,
        out_type=jax.ShapeDtypeStruct((NC, n, d), jnp.float32),
        mesh=mesh,
        scratch_types=[
            pltpu.VMEM((BLK, CHUNK), jnp.int32),
            pltpu.VMEM((BLK, CHUNK), jnp.int32),
            pltpu.VMEM((BLK * CHUNK,), jnp.float32),
            [pltpu.VMEM((CHUNK, d), jnp.float32) for _ in range(NBUF)],
            pltpu.VMEM_SHARED((n, d), jnp.float32),
            [pltpu.SemaphoreType.DMA for _ in range(NBUF)],
        ],
    )
    def k(support_hbm, src_hbm, dst_hbm, ew_hbm, out_hbm,
          src_v, dst_v, ew_v, rows, acc, gsem):
        c = lax.axis_index("c")
        s = lax.axis_index("s")
        wid = s * NC + c

        row0 = s * rows_per_sub
        for dd in range(d_regs):
            zsl = pl.ds(dd * LANES, LANES)

            def zrow(r, _):
                rows[0][r, zsl] = jnp.zeros((LANES,), jnp.float32)
                return _

            lax.fori_loop(0, CHUNK, zrow, None)

        def zcopy(i, _):
            pltpu.sync_copy(rows[0],
                            acc.at[pl.ds(row0 + i * CHUNK, CHUNK)])
            return _

        lax.fori_loop(0, rows_per_sub // CHUNK, zcopy, None)
        rem0 = rows_per_sub - (rows_per_sub // CHUNK) * CHUNK
        if rem0:
            pltpu.sync_copy(rows[0].at[pl.ds(0, rem0)],
                            acc.at[pl.ds(row0 + rows_per_sub - rem0, rem0)])
        if n_rem:
            @pl.when(s == 0)
            def _():
                pltpu.sync_copy(rows[0].at[pl.ds(0, n_rem)],
                                acc.at[pl.ds(NS * rows_per_sub, n_rem)])
        plsc.subcore_barrier()

        def gather(ci, buf, sem):
            pltpu.async_copy(support_hbm.at[src_v.at[ci]], buf, sem)

        def gather_wait(ci, buf, sem):
            pltpu.make_async_copy(support_hbm.at[src_v.at[ci]], buf,
                                  sem).wait()

        def scale(ci, buf):
            def group_body(g, _):
                w16 = ew_v[pl.ds(ci * CHUNK + g * LANES, LANES)]
                for j in range(LANES):
                    e = g * LANES + j
                    w = w16[j]
                    for dd in range(d_regs):
                        sl = pl.ds(dd * LANES, LANES)
                        buf[e, sl] = buf[e, sl] * w
                return _

            lax.fori_loop(0, CHUNK // LANES, group_body, None)

        def block_body(b, _):
            pltpu.sync_copy(src_hbm.at[wid, b], src_v)
            pltpu.sync_copy(dst_hbm.at[wid, b], dst_v)
            pltpu.sync_copy(ew_hbm.at[wid, b], ew_v)

            def chunk_body(ci, _):
                gather(ci, rows[0], gsem[0])
                gather_wait(ci, rows[0], gsem[0])
                scale(ci, rows[0])
                pltpu.sync_copy(rows[0], acc.at[dst_v.at[ci]], add=True)
                return _

            lax.fori_loop(0, BLK, chunk_body, None)
            return _

        lax.fori_loop(0, nblk, block_body, None)
        plsc.subcore_barrier()

        pltpu.sync_copy(acc.at[pl.ds(row0, rows_per_sub)],
                        out_hbm.at[c, pl.ds(row0, rows_per_sub)])
        if n_rem:
            @pl.when(s == 0)
            def _():
                pltpu.sync_copy(acc.at[pl.ds(NS * rows_per_sub, n_rem)],
                                out_hbm.at[c, pl.ds(NS * rows_per_sub, n_rem)])

    return k(support, src4, dst4, ew3)


def kernel(x, edge_index, edge_weight, W):
    n, _ = x.shape
    d = W.shape[1]
    e = edge_weight.shape[0]
    quantum = NW * BLK * CHUNK
    e_pad = ((e + quantum - 1) // quantum) * quantum
    pad = e_pad - e
    epw = e_pad // NW
    nblk = epw // (BLK * CHUNK)

    support = _matmul(x, W)

    src = jnp.pad(edge_index[0], (0, pad))
    dst = jnp.pad(edge_index[1], (0, pad))
    ew = jnp.pad(edge_weight, (0, pad))
    src4 = src.reshape(NW, nblk, BLK, CHUNK)
    dst4 = dst.reshape(NW, nblk, BLK, CHUNK)
    ew3 = ew.reshape(NW, nblk, BLK * CHUNK)

    partials = _sc_spmm(support, src4, dst4, ew3)
    return _combine(partials)


# speedup vs baseline: 2.9233x; 2.9233x over previous
import functools

import jax
import jax.numpy as jnp
from jax import lax
from jax.experimental import pallas as pl
from jax.experimental.pallas import tpu as pltpu
from jax.experimental.pallas import tpu_sc as plsc

NC = 2
NS = 16
NW = NC * NS
CHUNK = 80
BLK = 16
NBUF = 2
LANES = 16


def _matmul(x, W):
    n, d_in = x.shape
    d_out = W.shape[1]
    bm = 1000
    grid = (n // bm,)

    def body(x_ref, w_ref, o_ref):
        o_ref[...] = jnp.dot(x_ref[...], w_ref[...],
                             preferred_element_type=jnp.float32)

    return pl.pallas_call(
        body,
        grid=grid,
        in_specs=[
            pl.BlockSpec((bm, d_in), lambda i: (i, 0)),
            pl.BlockSpec((d_in, d_out), lambda i: (0, 0)),
        ],
        out_specs=pl.BlockSpec((bm, d_out), lambda i: (i, 0)),
        out_shape=jax.ShapeDtypeStruct((n, d_out), jnp.float32),
    )(x, W)


def _combine(partials):
    _, n, d = partials.shape
    bm = 1000
    grid = (n // bm,)

    def body(p_ref, o_ref):
        o_ref[...] = p_ref[0] + p_ref[1]

    return pl.pallas_call(
        body,
        grid=grid,
        in_specs=[pl.BlockSpec((2, bm, d), lambda i: (0, i, 0))],
        out_specs=pl.BlockSpec((bm, d), lambda i: (i, 0)),
        out_shape=jax.ShapeDtypeStruct((n, d), jnp.float32),
    )(partials)


def _sc_spmm(support, src4, dst4, ew3):
    n, d = support.shape
    nblk = src4.shape[1]
    rows_per_sub = (n // (8 * NS)) * 8
    n_rem = n - NS * rows_per_sub
    d_regs = d // LANES

    mesh = plsc.VectorSubcoreMesh(core_axis_name="c", subcore_axis_name="s")

    @functools.partial(
        pl.kernel,
        out_type=jax.ShapeDtypeStruct((NC, n, d), jnp.float32),
        mesh=mesh,
        scratch_types=[
            pltpu.VMEM((BLK, CHUNK), jnp.int32),
            pltpu.VMEM((BLK, CHUNK), jnp.int32),
            pltpu.VMEM((BLK * CHUNK,), jnp.float32),
            [pltpu.VMEM((CHUNK, d), jnp.float32) for _ in range(NBUF)],
            pltpu.VMEM_SHARED((n, d), jnp.float32),
            [pltpu.SemaphoreType.DMA for _ in range(NBUF)],
        ],
    )
    def k(support_hbm, src_hbm, dst_hbm, ew_hbm, out_hbm,
          src_v, dst_v, ew_v, rows, acc, gsem):
        c = lax.axis_index("c")
        s = lax.axis_index("s")
        wid = s * NC + c

        row0 = s * rows_per_sub
        for dd in range(d_regs):
            zsl = pl.ds(dd * LANES, LANES)

            def zrow(r, _):
                rows[0][r, zsl] = jnp.zeros((LANES,), jnp.float32)
                return _

            lax.fori_loop(0, CHUNK, zrow, None)

        def zcopy(i, _):
            pltpu.sync_copy(rows[0],
                            acc.at[pl.ds(row0 + i * CHUNK, CHUNK)])
            return _

        lax.fori_loop(0, rows_per_sub // CHUNK, zcopy, None)
        rem0 = rows_per_sub - (rows_per_sub // CHUNK) * CHUNK
        if rem0:
            pltpu.sync_copy(rows[0].at[pl.ds(0, rem0)],
                            acc.at[pl.ds(row0 + rows_per_sub - rem0, rem0)])
        if n_rem:
            @pl.when(s == 0)
            def _():
                pltpu.sync_copy(rows[0].at[pl.ds(0, n_rem)],
                                acc.at[pl.ds(NS * rows_per_sub, n_rem)])
        plsc.subcore_barrier()

        def gather(ci, buf, sem):
            pltpu.async_copy(support_hbm.at[src_v.at[ci]], buf, sem)

        def gather_wait(ci, buf, sem):
            pltpu.make_async_copy(support_hbm.at[src_v.at[ci]], buf,
                                  sem).wait()

        def scale(ci, buf):
            def group_body(g, _):
                w16 = ew_v[pl.ds(ci * CHUNK + g * LANES, LANES)]
                for j in range(LANES):
                    e = g * LANES + j
                    w = w16[j]
                    for dd in range(d_regs):
                        sl = pl.ds(dd * LANES, LANES)
                        buf[e, sl] = buf[e, sl] * w
                return _

            lax.fori_loop(0, CHUNK // LANES, group_body, None)

        def block_body(b, _):
            pltpu.sync_copy(src_hbm.at[wid, b], src_v)
            pltpu.sync_copy(dst_hbm.at[wid, b], dst_v)
            pltpu.sync_copy(ew_hbm.at[wid, b], ew_v)

            gather(0, rows[0], gsem[0])

            def pair_body(qi, _):
                for r in range(2):
                    ci = qi * 2 + r

                    @pl.when(ci < BLK - 1)
                    def _():
                        gather(ci + 1, rows[1 - r], gsem[1 - r])

                    gather_wait(ci, rows[r], gsem[r])
                    scale(ci, rows[r])
                    pltpu.sync_copy(rows[r], acc.at[dst_v.at[ci]], add=True)
                return _

            lax.fori_loop(0, BLK // 2, pair_body, None)
            return _

        lax.fori_loop(0, nblk, block_body, None)
        plsc.subcore_barrier()

        pltpu.sync_copy(acc.at[pl.ds(row0, rows_per_sub)],
                        out_hbm.at[c, pl.ds(row0, rows_per_sub)])
        if n_rem:
            @pl.when(s == 0)
            def _():
                pltpu.sync_copy(acc.at[pl.ds(NS * rows_per_sub, n_rem)],
                                out_hbm.at[c, pl.ds(NS * rows_per_sub, n_rem)])

    return k(support, src4, dst4, ew3)


def kernel(x, edge_index, edge_weight, W):
    n, _ = x.shape
    d = W.shape[1]
    e = edge_weight.shape[0]
    quantum = NW * BLK * CHUNK
    e_pad = ((e + quantum - 1) // quantum) * quantum
    pad = e_pad - e
    epw = e_pad // NW
    nblk = epw // (BLK * CHUNK)

    support = _matmul(x, W)

    pad_idx = jnp.arange(pad, dtype=jnp.int32) % n
    src = jnp.concatenate([edge_index[0], pad_idx])
    dst = jnp.concatenate([edge_index[1], pad_idx])
    ew = jnp.pad(edge_weight, (0, pad))
    src4 = src.reshape(NW, nblk, BLK, CHUNK)
    dst4 = dst.reshape(NW, nblk, BLK, CHUNK)
    ew3 = ew.reshape(NW, nblk, BLK * CHUNK)

    partials = _sc_spmm(support, src4, dst4, ew3)
    return _combine(partials)


# speedup vs baseline: 3.2544x; 1.1133x over previous
import functools

import jax
import jax.numpy as jnp
from jax import lax
from jax.experimental import pallas as pl
from jax.experimental.pallas import tpu as pltpu
from jax.experimental.pallas import tpu_sc as plsc

NC = 2
NS = 16
NW = NC * NS
CHUNK = 80
BLK = 16
NBUF = 4
LANES = 16


def _matmul(x, W):
    n, d_in = x.shape
    d_out = W.shape[1]
    bm = 1000
    grid = (n // bm,)

    def body(x_ref, w_ref, o_ref):
        o_ref[...] = jnp.dot(x_ref[...], w_ref[...],
                             preferred_element_type=jnp.float32)

    return pl.pallas_call(
        body,
        grid=grid,
        in_specs=[
            pl.BlockSpec((bm, d_in), lambda i: (i, 0)),
            pl.BlockSpec((d_in, d_out), lambda i: (0, 0)),
        ],
        out_specs=pl.BlockSpec((bm, d_out), lambda i: (i, 0)),
        out_shape=jax.ShapeDtypeStruct((n, d_out), jnp.float32),
    )(x, W)


def _combine(partials):
    _, n, d = partials.shape
    bm = 1000
    grid = (n // bm,)

    def body(p_ref, o_ref):
        o_ref[...] = p_ref[0] + p_ref[1]

    return pl.pallas_call(
        body,
        grid=grid,
        in_specs=[pl.BlockSpec((2, bm, d), lambda i: (0, i, 0))],
        out_specs=pl.BlockSpec((bm, d), lambda i: (i, 0)),
        out_shape=jax.ShapeDtypeStruct((n, d), jnp.float32),
    )(partials)


def _sc_spmm(support, src4, dst4, ew3):
    n, d = support.shape
    nblk = src4.shape[1]
    rows_per_sub = (n // (8 * NS)) * 8
    n_rem = n - NS * rows_per_sub
    d_regs = d // LANES

    mesh = plsc.VectorSubcoreMesh(core_axis_name="c", subcore_axis_name="s")

    @functools.partial(
        pl.kernel,
        out_type=jax.ShapeDtypeStruct((NC, n, d), jnp.float32),
        mesh=mesh,
        scratch_types=[
            pltpu.VMEM((BLK, CHUNK), jnp.int32),
            pltpu.VMEM((BLK, CHUNK), jnp.int32),
            pltpu.VMEM((BLK * CHUNK,), jnp.float32),
            [pltpu.VMEM((CHUNK, d), jnp.float32) for _ in range(NBUF)],
            pltpu.VMEM_SHARED((n, d), jnp.float32),
            [pltpu.SemaphoreType.DMA for _ in range(NBUF)],
            [pltpu.SemaphoreType.DMA for _ in range(NBUF)],
        ],
    )
    def k(support_hbm, src_hbm, dst_hbm, ew_hbm, out_hbm,
          src_v, dst_v, ew_v, rows, acc, gsem, ssem):
        c = lax.axis_index("c")
        s = lax.axis_index("s")
        wid = s * NC + c

        row0 = s * rows_per_sub
        for dd in range(d_regs):
            zsl = pl.ds(dd * LANES, LANES)

            def zrow(r, _):
                rows[0][r, zsl] = jnp.zeros((LANES,), jnp.float32)
                return _

            lax.fori_loop(0, CHUNK, zrow, None)

        def zcopy(i, _):
            pltpu.sync_copy(rows[0],
                            acc.at[pl.ds(row0 + i * CHUNK, CHUNK)])
            return _

        lax.fori_loop(0, rows_per_sub // CHUNK, zcopy, None)
        rem0 = rows_per_sub - (rows_per_sub // CHUNK) * CHUNK
        if rem0:
            pltpu.sync_copy(rows[0].at[pl.ds(0, rem0)],
                            acc.at[pl.ds(row0 + rows_per_sub - rem0, rem0)])
        if n_rem:
            @pl.when(s == 0)
            def _():
                pltpu.sync_copy(rows[0].at[pl.ds(0, n_rem)],
                                acc.at[pl.ds(NS * rows_per_sub, n_rem)])
        plsc.subcore_barrier()

        def gather(ci, buf, sem):
            pltpu.async_copy(support_hbm.at[src_v.at[ci]], buf, sem)

        def gather_wait(ci, buf, sem):
            pltpu.make_async_copy(support_hbm.at[src_v.at[ci]], buf,
                                  sem).wait()

        def scatter(ci, buf, sem):
            pltpu.async_copy(buf, acc.at[dst_v.at[ci]], sem, add=True)

        def scatter_wait(ci, buf, sem):
            pltpu.make_async_copy(buf, acc.at[dst_v.at[ci]], sem).wait()

        def scale(ci, buf):
            def group_body(g, _):
                w16 = ew_v[pl.ds(ci * CHUNK + g * LANES, LANES)]
                for j in range(LANES):
                    e = g * LANES + j
                    w = w16[j]
                    for dd in range(d_regs):
                        sl = pl.ds(dd * LANES, LANES)
                        buf[e, sl] = buf[e, sl] * w
                return _

            lax.fori_loop(0, CHUNK // LANES, group_body, None)

        def block_body(b, _):
            pltpu.sync_copy(src_hbm.at[wid, b], src_v)
            pltpu.sync_copy(dst_hbm.at[wid, b], dst_v)
            pltpu.sync_copy(ew_hbm.at[wid, b], ew_v)

            gather(0, rows[0], gsem[0])
            gather(1, rows[1], gsem[1])

            def quad_body(qi, _):
                for r in range(NBUF):
                    ci = qi * NBUF + r
                    r2 = (r + 2) % NBUF

                    @pl.when(jnp.logical_and(ci >= 2, ci <= BLK - 3))
                    def _():
                        scatter_wait(ci - 2, rows[r2], ssem[r2])
                        gather(ci + 2, rows[r2], gsem[r2])

                    @pl.when(ci < 2)
                    def _():
                        gather(ci + 2, rows[r2], gsem[r2])

                    gather_wait(ci, rows[r], gsem[r])
                    scale(ci, rows[r])
                    scatter(ci, rows[r], ssem[r])
                return _

            lax.fori_loop(0, BLK // NBUF, quad_body, None)

            for r in range(NBUF):
                ci = BLK - NBUF + r
                scatter_wait(ci, rows[r], ssem[r])
            return _

        lax.fori_loop(0, nblk, block_body, None)
        plsc.subcore_barrier()

        pltpu.sync_copy(acc.at[pl.ds(row0, rows_per_sub)],
                        out_hbm.at[c, pl.ds(row0, rows_per_sub)])
        if n_rem:
            @pl.when(s == 0)
            def _():
                pltpu.sync_copy(acc.at[pl.ds(NS * rows_per_sub, n_rem)],
                                out_hbm.at[c, pl.ds(NS * rows_per_sub, n_rem)])

    return k(support, src4, dst4, ew3)


def kernel(x, edge_index, edge_weight, W):
    n, _ = x.shape
    d = W.shape[1]
    e = edge_weight.shape[0]
    quantum = NW * BLK * CHUNK
    e_pad = ((e + quantum - 1) // quantum) * quantum
    pad = e_pad - e
    epw = e_pad // NW
    nblk = epw // (BLK * CHUNK)

    support = _matmul(x, W)

    pad_idx = jnp.arange(pad, dtype=jnp.int32) % n
    src = jnp.concatenate([edge_index[0], pad_idx])
    dst = jnp.concatenate([edge_index[1], pad_idx])
    ew = jnp.pad(edge_weight, (0, pad))
    src4 = src.reshape(NW, nblk, BLK, CHUNK)
    dst4 = dst.reshape(NW, nblk, BLK, CHUNK)
    ew3 = ew.reshape(NW, nblk, BLK * CHUNK)

    partials = _sc_spmm(support, src4, dst4, ew3)
    return _combine(partials)


# speedup vs baseline: 3.2960x; 1.0128x over previous
import functools

import jax
import jax.numpy as jnp
from jax import lax
from jax.experimental import pallas as pl
from jax.experimental.pallas import tpu as pltpu
from jax.experimental.pallas import tpu_sc as plsc

NC = 2
NS = 16
NW = NC * NS
CHUNK = 80
BLK = 16
NBUF = 4
LANES = 16


def _matmul(x, W):
    n, d_in = x.shape
    d_out = W.shape[1]
    bm = 2000
    grid = (n // bm,)

    def body(x_ref, w_ref, o_ref):
        o_ref[...] = jnp.dot(x_ref[...], w_ref[...],
                             preferred_element_type=jnp.float32)

    return pl.pallas_call(
        body,
        grid=grid,
        in_specs=[
            pl.BlockSpec((bm, d_in), lambda i: (i, 0)),
            pl.BlockSpec((d_in, d_out), lambda i: (0, 0)),
        ],
        out_specs=pl.BlockSpec((bm, d_out), lambda i: (i, 0)),
        out_shape=jax.ShapeDtypeStruct((n, d_out), jnp.float32),
    )(x, W)


def _combine(partials):
    _, n, d = partials.shape
    bm = 1000
    grid = (n // bm,)

    def body(p_ref, o_ref):
        o_ref[...] = p_ref[0] + p_ref[1]

    return pl.pallas_call(
        body,
        grid=grid,
        in_specs=[pl.BlockSpec((2, bm, d), lambda i: (0, i, 0))],
        out_specs=pl.BlockSpec((bm, d), lambda i: (i, 0)),
        out_shape=jax.ShapeDtypeStruct((n, d), jnp.float32),
    )(partials)


def _sc_spmm(support, src4, dst4, ew3):
    n, d = support.shape
    nblk = src4.shape[1]
    rows_per_sub = (n // (8 * NS)) * 8
    n_rem = n - NS * rows_per_sub
    d_regs = d // LANES

    mesh = plsc.VectorSubcoreMesh(core_axis_name="c", subcore_axis_name="s")

    @functools.partial(
        pl.kernel,
        out_type=jax.ShapeDtypeStruct((NC, n, d), jnp.float32),
        mesh=mesh,
        scratch_types=[
            pltpu.VMEM((BLK, CHUNK), jnp.int32),
            pltpu.VMEM((BLK, CHUNK), jnp.int32),
            pltpu.VMEM((BLK * CHUNK,), jnp.float32),
            [pltpu.VMEM((CHUNK, d), jnp.float32) for _ in range(NBUF)],
            pltpu.VMEM_SHARED((n, d), jnp.float32),
            [pltpu.SemaphoreType.DMA for _ in range(NBUF)],
            [pltpu.SemaphoreType.DMA for _ in range(NBUF)],
        ],
    )
    def k(support_hbm, src_hbm, dst_hbm, ew_hbm, out_hbm,
          src_v, dst_v, ew_v, rows, acc, gsem, ssem):
        c = lax.axis_index("c")
        s = lax.axis_index("s")
        wid = s * NC + c

        row0 = s * rows_per_sub
        for dd in range(d_regs):
            zsl = pl.ds(dd * LANES, LANES)

            def zrow(r, _):
                rows[0][r, zsl] = jnp.zeros((LANES,), jnp.float32)
                return _

            lax.fori_loop(0, CHUNK, zrow, None)

        def zcopy(i, _):
            pltpu.sync_copy(rows[0],
                            acc.at[pl.ds(row0 + i * CHUNK, CHUNK)])
            return _

        lax.fori_loop(0, rows_per_sub // CHUNK, zcopy, None)
        rem0 = rows_per_sub - (rows_per_sub // CHUNK) * CHUNK
        if rem0:
            pltpu.sync_copy(rows[0].at[pl.ds(0, rem0)],
                            acc.at[pl.ds(row0 + rows_per_sub - rem0, rem0)])
        if n_rem:
            @pl.when(s == 0)
            def _():
                pltpu.sync_copy(rows[0].at[pl.ds(0, n_rem)],
                                acc.at[pl.ds(NS * rows_per_sub, n_rem)])
        plsc.subcore_barrier()

        def gather(ci, buf, sem):
            pltpu.async_copy(support_hbm.at[src_v.at[ci]], buf, sem)

        def gather_wait(ci, buf, sem):
            pltpu.make_async_copy(support_hbm.at[src_v.at[ci]], buf,
                                  sem).wait()

        def scatter(ci, buf, sem):
            pltpu.async_copy(buf, acc.at[dst_v.at[ci]], sem, add=True)

        def scatter_wait(ci, buf, sem):
            pltpu.make_async_copy(buf, acc.at[dst_v.at[ci]], sem).wait()

        def scale(ci, buf):
            def group_body(g, _):
                w16 = ew_v[pl.ds(ci * CHUNK + g * LANES, LANES)]
                for j in range(LANES):
                    e = g * LANES + j
                    w = w16[j]
                    for dd in range(d_regs):
                        sl = pl.ds(dd * LANES, LANES)
                        buf[e, sl] = buf[e, sl] * w
                return _

            lax.fori_loop(0, CHUNK // LANES, group_body, None)

        def block_body(b, _):
            pltpu.sync_copy(src_hbm.at[wid, b], src_v)
            pltpu.sync_copy(dst_hbm.at[wid, b], dst_v)
            pltpu.sync_copy(ew_hbm.at[wid, b], ew_v)

            gather(0, rows[0], gsem[0])
            gather(1, rows[1], gsem[1])

            def quad_body(qi, _):
                for r in range(NBUF):
                    ci = qi * NBUF + r
                    r2 = (r + 2) % NBUF

                    @pl.when(jnp.logical_and(ci >= 2, ci <= BLK - 3))
                    def _():
                        scatter_wait(ci - 2, rows[r2], ssem[r2])
                        gather(ci + 2, rows[r2], gsem[r2])

                    @pl.when(ci < 2)
                    def _():
                        gather(ci + 2, rows[r2], gsem[r2])

                    gather_wait(ci, rows[r], gsem[r])
                    scale(ci, rows[r])
                    scatter(ci, rows[r], ssem[r])
                return _

            lax.fori_loop(0, BLK // NBUF, quad_body, None)

            for r in range(NBUF):
                ci = BLK - NBUF + r
                scatter_wait(ci, rows[r], ssem[r])
            return _

        lax.fori_loop(0, nblk, block_body, None)
        plsc.subcore_barrier()

        pltpu.sync_copy(acc.at[pl.ds(row0, rows_per_sub)],
                        out_hbm.at[c, pl.ds(row0, rows_per_sub)])
        if n_rem:
            @pl.when(s == 0)
            def _():
                pltpu.sync_copy(acc.at[pl.ds(NS * rows_per_sub, n_rem)],
                                out_hbm.at[c, pl.ds(NS * rows_per_sub, n_rem)])

    return k(support, src4, dst4, ew3)


def kernel(x, edge_index, edge_weight, W):
    n, _ = x.shape
    d = W.shape[1]
    e = edge_weight.shape[0]
    quantum = NW * BLK * CHUNK
    e_pad = ((e + quantum - 1) // quantum) * quantum
    pad = e_pad - e
    epw = e_pad // NW
    nblk = epw // (BLK * CHUNK)

    support = _matmul(x, W)

    pad_idx = jnp.arange(pad, dtype=jnp.int32) % n
    src = jnp.concatenate([edge_index[0], pad_idx])
    dst = jnp.concatenate([edge_index[1], pad_idx])
    ew = jnp.pad(edge_weight, (0, pad))
    src4 = src.reshape(NW, nblk, BLK, CHUNK)
    dst4 = dst.reshape(NW, nblk, BLK, CHUNK)
    ew3 = ew.reshape(NW, nblk, BLK * CHUNK)

    partials = _sc_spmm(support, src4, dst4, ew3)
    return _combine(partials)
